# baseline (speedup 1.0000x reference)
import functools

import jax
import jax.numpy as jnp
from jax import lax
from jax.experimental import pallas as pl
from jax.experimental.pallas import tpu as pltpu
from jax.experimental.pallas import tpu_sc as plsc

B = 16384
N_WORDS = 327680
H = 50
NSLOT = 6
NCHAR = 5
NCOL = NSLOT * 3
CID_PAD = 8

NW = 32
B_PER_W = B // NW
PAIRS_PER_W = B_PER_W * NSLOT
CHUNK = 128
NCHUNK = PAIRS_PER_W // CHUNK
GROUPS = B_PER_W // 16


def _fold_table_kernel(ct_ref, wr_ref, br_ref, out_ref):
    out_ref[...] = (
        jnp.dot(ct_ref[...], wr_ref[...], preferred_element_type=jnp.float32)
        + br_ref[...]
    )


def _build_fold_table(char_table, W, b):
    w_r = W.reshape(NSLOT, H, 3).transpose(1, 0, 2).reshape(H, NCOL)
    ct_p = jnp.zeros((512, 64), jnp.float32).at[:500, :H].set(char_table)
    wr_p = jnp.zeros((64, 128), jnp.float32).at[:H, :NCOL].set(w_r)
    br = jnp.tile(b, NSLOT) / (NSLOT * NCHAR)
    br_p = jnp.zeros((1, 128), jnp.float32).at[0, :NCOL].set(br)
    t_full = pl.pallas_call(
        _fold_table_kernel,
        out_shape=jax.ShapeDtypeStruct((512, 128), jnp.float32),
    )(ct_p, wr_p, br_p)
    return t_full[:500, :NCOL].reshape(-1)


W_ROWS = N_WORDS // NW


def _pad_kernel(src_hbm, dst_hbm, buf):
    nc = 2
    wid = lax.axis_index("s") * nc + lax.axis_index("c")
    r0 = wid * W_ROWS
    pltpu.sync_copy(src_hbm.at[pl.ds(r0, W_ROWS)], buf)
    pltpu.sync_copy(buf, dst_hbm.at[pl.ds(r0, W_ROWS), pl.ds(0, NCHAR)])


def _sc_kernel(idx_hbm, cid_hbm, t_hbm, out_hbm, idx_v, cid_v, t_v, out_v, sem):
    nc = 2
    wid = lax.axis_index("s") * nc + lax.axis_index("c")

    pltpu.sync_copy(idx_hbm.at[pl.ds(wid * NCHUNK, NCHUNK)], idx_v)

    copies = []
    for q in range(NCHUNK):
        copies.append(
            pltpu.async_copy(
                cid_hbm.at[idx_v.at[q]],
                cid_v.at[pl.ds(q * CHUNK, CHUNK)],
                sem,
            )
        )
    pltpu.sync_copy(t_hbm, t_v)
    for c in copies:
        c.wait()

    lanes = lax.iota(jnp.int32, 16)

    def body(g, _):
        accs = [jnp.zeros((16,), jnp.float32) for _ in range(3)]
        for j in range(NSLOT):
            rowvec = lanes * NSLOT + (g * (16 * NSLOT) + j)
            for c in range(NCHAR):
                cid = plsc.load_gather(cid_v, [rowvec, jnp.full((16,), c, jnp.int32)])
                taddr = cid * NCOL
                for k in range(3):
                    accs[k] = accs[k] + plsc.load_gather(t_v, [taddr + (j * 3 + k)])
        m = jnp.maximum(accs[0], jnp.maximum(accs[1], accs[2]))
        outrow = g * 16 + lanes
        for k in range(3):
            plsc.store_scatter(
                out_v, [outrow, jnp.full((16,), k, jnp.int32)], jnp.exp(accs[k] - m)
            )
        return 0

    lax.fori_loop(0, GROUPS, body, 0)

    pltpu.sync_copy(out_v, out_hbm.at[pl.ds(wid * B_PER_W, B_PER_W)])


@jax.jit
def kernel(char_ids, buffer_idx, stack_idx, char_table, W, b):
    t_tab = _build_fold_table(char_table, W, b)
    mesh = plsc.VectorSubcoreMesh(core_axis_name="c", subcore_axis_name="s")
    sc_params = pltpu.CompilerParams(
        needs_layout_passes=False, use_tc_tiling_on_sc=False
    )
    pad_run = functools.partial(
        pl.kernel,
        mesh=mesh,
        out_type=jax.ShapeDtypeStruct((N_WORDS, CID_PAD), jnp.int32),
        scratch_types=[pltpu.VMEM((W_ROWS, NCHAR), jnp.int32)],
        compiler_params=sc_params,
    )(_pad_kernel)
    cid_p = pad_run(char_ids.astype(jnp.int32))
    idx_r = (
        jnp.concatenate(
            [buffer_idx.astype(jnp.int32), stack_idx.astype(jnp.int32)], axis=1
        ).reshape(NW * NCHUNK, CHUNK)
    )

    run = functools.partial(
        pl.kernel,
        mesh=mesh,
        out_type=jax.ShapeDtypeStruct((B, 3), jnp.float32),
        scratch_types=[
            pltpu.VMEM((NCHUNK, CHUNK), jnp.int32),
            pltpu.VMEM((PAIRS_PER_W, CID_PAD), jnp.int32),
            pltpu.VMEM((500 * NCOL,), jnp.float32),
            pltpu.VMEM((B_PER_W, 3), jnp.float32),
            pltpu.SemaphoreType.DMA,
        ],
        compiler_params=sc_params,
    )(_sc_kernel)
    return run(idx_r, cid_p, t_tab)


# speedup vs baseline: 2.5909x; 2.5909x over previous
import functools

import jax
import jax.numpy as jnp
from jax import lax
from jax.experimental import pallas as pl
from jax.experimental.pallas import tpu as pltpu
from jax.experimental.pallas import tpu_sc as plsc

B = 16384
N_WORDS = 327680
H = 50
NSLOT = 6
NCHAR = 5
NCOL = NSLOT * 3
CID_PAD = 8

NW = 32
B_PER_W = B // NW
PAIRS_PER_W = B_PER_W * NSLOT
CHUNK = 128
NCHUNK = PAIRS_PER_W // CHUNK
GROUPS = B_PER_W // 16


def _fold_table_kernel(ct_ref, wr_ref, br_ref, out_ref):
    out_ref[...] = (
        jnp.dot(ct_ref[...], wr_ref[...], preferred_element_type=jnp.float32)
        + br_ref[...]
    )


def _build_fold_table(char_table, W, b):
    w_r = W.reshape(NSLOT, H, 3).transpose(1, 0, 2).reshape(H, NCOL)
    ct_p = jnp.zeros((512, 64), jnp.float32).at[:500, :H].set(char_table)
    wr_p = jnp.zeros((64, 128), jnp.float32).at[:H, :NCOL].set(w_r)
    br = jnp.tile(b, NSLOT) / (NSLOT * NCHAR)
    br_p = jnp.zeros((1, 128), jnp.float32).at[0, :NCOL].set(br)
    t_full = pl.pallas_call(
        _fold_table_kernel,
        out_shape=jax.ShapeDtypeStruct((512, 128), jnp.float32),
    )(ct_p, wr_p, br_p)
    return t_full[:500, :NCOL].reshape(-1)


W_ROWS = N_WORDS // NW


def _pad_kernel(src_hbm, dst_hbm, buf):
    nc = 2
    wid = lax.axis_index("s") * nc + lax.axis_index("c")
    r0 = wid * W_ROWS
    pltpu.sync_copy(src_hbm.at[pl.ds(r0, W_ROWS)], buf.at[:, pl.ds(0, NCHAR)])
    pltpu.sync_copy(buf, dst_hbm.at[pl.ds(r0, W_ROWS)])


def _sc_kernel(idx_hbm, cid_hbm, t_hbm, out_hbm, idx_v, cid_v, t_v, out_v, sem):
    nc = 2
    wid = lax.axis_index("s") * nc + lax.axis_index("c")

    pltpu.sync_copy(idx_hbm.at[pl.ds(wid * NCHUNK, NCHUNK)], idx_v)

    copies = []
    for q in range(NCHUNK):
        copies.append(
            pltpu.async_copy(
                cid_hbm.at[idx_v.at[q]],
                cid_v.at[pl.ds(q * CHUNK, CHUNK)],
                sem,
            )
        )
    pltpu.sync_copy(t_hbm, t_v)
    for c in copies:
        c.wait()

    lanes = lax.iota(jnp.int32, 16)

    def body(g, _):
        accs = [jnp.zeros((16,), jnp.float32) for _ in range(3)]
        for j in range(NSLOT):
            rowvec = lanes * NSLOT + (g * (16 * NSLOT) + j)
            for c in range(NCHAR):
                cid = plsc.load_gather(cid_v, [rowvec, jnp.full((16,), c, jnp.int32)])
                taddr = cid * NCOL
                for k in range(3):
                    accs[k] = accs[k] + plsc.load_gather(t_v, [taddr + (j * 3 + k)])
        m = jnp.maximum(accs[0], jnp.maximum(accs[1], accs[2]))
        outrow = g * 16 + lanes
        for k in range(3):
            plsc.store_scatter(
                out_v, [outrow, jnp.full((16,), k, jnp.int32)], jnp.exp(accs[k] - m)
            )
        return 0

    lax.fori_loop(0, GROUPS, body, 0)

    pltpu.sync_copy(out_v, out_hbm.at[pl.ds(wid * B_PER_W, B_PER_W)])


@jax.jit
def kernel(char_ids, buffer_idx, stack_idx, char_table, W, b):
    t_tab = _build_fold_table(char_table, W, b)
    mesh = plsc.VectorSubcoreMesh(core_axis_name="c", subcore_axis_name="s")
    sc_params = pltpu.CompilerParams(
        needs_layout_passes=False, use_tc_tiling_on_sc=False
    )
    pad_run = functools.partial(
        pl.kernel,
        mesh=mesh,
        out_type=jax.ShapeDtypeStruct((N_WORDS, CID_PAD), jnp.int32),
        scratch_types=[pltpu.VMEM((W_ROWS, CID_PAD), jnp.int32)],
        compiler_params=sc_params,
    )(_pad_kernel)
    cid_p = pad_run(char_ids.astype(jnp.int32))
    idx_r = (
        jnp.concatenate(
            [buffer_idx.astype(jnp.int32), stack_idx.astype(jnp.int32)], axis=1
        ).reshape(NW * NCHUNK, CHUNK)
    )

    run = functools.partial(
        pl.kernel,
        mesh=mesh,
        out_type=jax.ShapeDtypeStruct((B, 3), jnp.float32),
        scratch_types=[
            pltpu.VMEM((NCHUNK, CHUNK), jnp.int32),
            pltpu.VMEM((PAIRS_PER_W, CID_PAD), jnp.int32),
            pltpu.VMEM((500 * NCOL,), jnp.float32),
            pltpu.VMEM((B_PER_W, 3), jnp.float32),
            pltpu.SemaphoreType.DMA,
        ],
        compiler_params=sc_params,
    )(_sc_kernel)
    return run(idx_r, cid_p, t_tab)


# speedup vs baseline: 9.7790x; 3.7744x over previous
import functools

import jax
import jax.numpy as jnp
from jax import lax
from jax.experimental import pallas as pl
from jax.experimental.pallas import tpu as pltpu
from jax.experimental.pallas import tpu_sc as plsc

B = 16384
N_WORDS = 327680
H = 50
NSLOT = 6
NCHAR = 5
NCOL = NSLOT * 3

NW = 32
B_PER_W = B // NW
PAIRS_PER_W = B_PER_W * NSLOT
CHUNK = 128
NCHUNK = PAIRS_PER_W // CHUNK
ROWS_PER_SLOT = B_PER_W // CHUNK
NGATHER = NCHAR * NCHUNK
DEPTH = 16
GROUPS = B_PER_W // 16


def _fold_table_kernel(ct_ref, wr_ref, br_ref, out_ref):
    out_ref[...] = (
        jnp.dot(ct_ref[...], wr_ref[...], preferred_element_type=jnp.float32)
        + br_ref[...]
    )


def _build_fold_table(char_table, W, b):
    w_r = W.reshape(NSLOT, H, 3).transpose(1, 0, 2).reshape(H, NCOL)
    ct_p = jnp.zeros((512, 64), jnp.float32).at[:500, :H].set(char_table)
    wr_p = jnp.zeros((64, 128), jnp.float32).at[:H, :NCOL].set(w_r)
    br = jnp.tile(b, NSLOT) / (NSLOT * NCHAR)
    br_p = jnp.zeros((1, 128), jnp.float32).at[0, :NCOL].set(br)
    t_full = pl.pallas_call(
        _fold_table_kernel,
        out_shape=jax.ShapeDtypeStruct((512, 128), jnp.float32),
    )(ct_p, wr_p, br_p)
    return t_full[:500, :NCOL].reshape(-1)


def _sc_kernel(bidx_hbm, sidx_hbm, cid_hbm, t_hbm, out_hbm,
               idx_v, gidx_v, cid_v, t_v, out_v, sem):
    nc = 2
    wid = lax.axis_index("s") * nc + lax.axis_index("c")

    for j in range(3):
        pltpu.sync_copy(
            bidx_hbm.at[pl.ds(j * CHUNK + wid * ROWS_PER_SLOT, ROWS_PER_SLOT)],
            idx_v.at[pl.ds(j * ROWS_PER_SLOT, ROWS_PER_SLOT)],
        )
        pltpu.sync_copy(
            sidx_hbm.at[pl.ds(j * CHUNK + wid * ROWS_PER_SLOT, ROWS_PER_SLOT)],
            idx_v.at[pl.ds((j + 3) * ROWS_PER_SLOT, ROWS_PER_SLOT)],
        )

    def bld(q, _):
        for v in range(CHUNK // 16):
            base = idx_v[q, pl.ds(v * 16, 16)]
            for c in range(NCHAR):
                gidx_v[c * NCHUNK + q, pl.ds(v * 16, 16)] = base + jnp.full(
                    (16,), c * N_WORDS, jnp.int32
                )
        return 0

    lax.fori_loop(0, NCHUNK, bld, 0)

    pltpu.sync_copy(t_hbm, t_v)

    def start(r):
        return pltpu.async_copy(
            cid_hbm.at[gidx_v.at[r]], cid_v.at[pl.ds(r * CHUNK, CHUNK)], sem
        )

    for r in range(DEPTH):
        start(r)

    def fire(r, _):
        start(r)
        pltpu.make_async_copy(
            cid_hbm.at[gidx_v.at[r - DEPTH]],
            cid_v.at[pl.ds((r - DEPTH) * CHUNK, CHUNK)],
            sem,
        ).wait()
        return 0

    lax.fori_loop(DEPTH, NGATHER, fire, 0)
    for r in range(NGATHER - DEPTH, NGATHER):
        pltpu.make_async_copy(
            cid_hbm.at[gidx_v.at[r]], cid_v.at[pl.ds(r * CHUNK, CHUNK)], sem
        ).wait()

    lanes = lax.iota(jnp.int32, 16)

    def body(g, _):
        accs = [jnp.zeros((16,), jnp.float32) for _ in range(3)]
        for j in range(NSLOT):
            for c in range(NCHAR):
                cid = cid_v[pl.ds(c * PAIRS_PER_W + j * B_PER_W + g * 16, 16)]
                taddr = cid * NCOL
                for k in range(3):
                    accs[k] = accs[k] + plsc.load_gather(t_v, [taddr + (j * 3 + k)])
        m = jnp.maximum(accs[0], jnp.maximum(accs[1], accs[2]))
        outrow = g * 16 + lanes
        for k in range(3):
            plsc.store_scatter(
                out_v, [outrow, jnp.full((16,), k, jnp.int32)], jnp.exp(accs[k] - m)
            )
        return 0

    lax.fori_loop(0, GROUPS, body, 0)

    pltpu.sync_copy(out_v, out_hbm.at[pl.ds(wid * B_PER_W, B_PER_W)])


@jax.jit
def kernel(char_ids, buffer_idx, stack_idx, char_table, W, b):
    t_tab = _build_fold_table(char_table, W, b)
    cid_flat = char_ids.astype(jnp.int32).T.reshape(-1)
    bidx_r = buffer_idx.astype(jnp.int32).T.reshape(3 * B // CHUNK, CHUNK)
    sidx_r = stack_idx.astype(jnp.int32).T.reshape(3 * B // CHUNK, CHUNK)

    mesh = plsc.VectorSubcoreMesh(core_axis_name="c", subcore_axis_name="s")
    sc_params = pltpu.CompilerParams(
        needs_layout_passes=False, use_tc_tiling_on_sc=False
    )
    run = functools.partial(
        pl.kernel,
        mesh=mesh,
        out_type=jax.ShapeDtypeStruct((B, 3), jnp.float32),
        scratch_types=[
            pltpu.VMEM((NCHUNK, CHUNK), jnp.int32),
            pltpu.VMEM((NGATHER, CHUNK), jnp.int32),
            pltpu.VMEM((NCHAR * PAIRS_PER_W,), jnp.int32),
            pltpu.VMEM((500 * NCOL,), jnp.float32),
            pltpu.VMEM((B_PER_W, 3), jnp.float32),
            pltpu.SemaphoreType.DMA,
        ],
        compiler_params=sc_params,
    )(_sc_kernel)
    return run(bidx_r, sidx_r, cid_flat, t_tab)


# speedup vs baseline: 10.5482x; 1.0787x over previous
import functools

import jax
import jax.numpy as jnp
from jax import lax
from jax.experimental import pallas as pl
from jax.experimental.pallas import tpu as pltpu
from jax.experimental.pallas import tpu_sc as plsc

B = 16384
N_WORDS = 327680
H = 50
NSLOT = 6
NCHAR = 5
NCOL = NSLOT * 3

NW = 32
B_PER_W = B // NW
PAIRS_PER_W = B_PER_W * NSLOT
CHUNK = 128
NCHUNK = PAIRS_PER_W // CHUNK
ROWS_PER_SLOT = B_PER_W // CHUNK
NPLANE = 2
NGATHER = NPLANE * NCHUNK
DEPTH = 16
GROUPS = B_PER_W // 16


def _fold_table_kernel(ct_ref, wr_ref, br_ref, out_ref):
    out_ref[...] = (
        jnp.dot(ct_ref[...], wr_ref[...], preferred_element_type=jnp.float32)
        + br_ref[...]
    )


def _build_fold_table(char_table, W, b):
    w_r = W.reshape(NSLOT, H, 3).transpose(1, 0, 2).reshape(H, NCOL)
    ct_p = jnp.zeros((512, 64), jnp.float32).at[:500, :H].set(char_table)
    wr_p = jnp.zeros((64, 128), jnp.float32).at[:H, :NCOL].set(w_r)
    br = jnp.tile(b, NSLOT) / (NSLOT * NCHAR)
    br_p = jnp.zeros((1, 128), jnp.float32).at[0, :NCOL].set(br)
    t_full = pl.pallas_call(
        _fold_table_kernel,
        out_shape=jax.ShapeDtypeStruct((512, 128), jnp.float32),
    )(ct_p, wr_p, br_p)
    return t_full[:500, :NCOL].reshape(-1)


def _sc_kernel(bidx_hbm, sidx_hbm, p0_hbm, p1_hbm, t_hbm, out_hbm,
               idx_v, cid_v, t_v, out_v, sem):
    nc = 2
    wid = lax.axis_index("s") * nc + lax.axis_index("c")

    for j in range(3):
        pltpu.sync_copy(
            bidx_hbm.at[pl.ds(j * CHUNK + wid * ROWS_PER_SLOT, ROWS_PER_SLOT)],
            idx_v.at[pl.ds(j * ROWS_PER_SLOT, ROWS_PER_SLOT)],
        )
        pltpu.sync_copy(
            sidx_hbm.at[pl.ds(j * CHUNK + wid * ROWS_PER_SLOT, ROWS_PER_SLOT)],
            idx_v.at[pl.ds((j + 3) * ROWS_PER_SLOT, ROWS_PER_SLOT)],
        )

    pltpu.sync_copy(t_hbm, t_v)

    def start(r):
        plane, q = divmod(r, NCHUNK)
        src = p0_hbm if plane == 0 else p1_hbm
        return pltpu.async_copy(
            src.at[idx_v.at[q]],
            cid_v.at[pl.ds(r * CHUNK, CHUNK)],
            sem,
        )

    def drain(r):
        plane, q = divmod(r, NCHUNK)
        src = p0_hbm if plane == 0 else p1_hbm
        pltpu.make_async_copy(
            src.at[idx_v.at[q]], cid_v.at[pl.ds(r * CHUNK, CHUNK)], sem
        ).wait()

    for r in range(DEPTH):
        start(r)
    for r in range(DEPTH, NGATHER):
        start(r)
        drain(r - DEPTH)
    for r in range(NGATHER - DEPTH, NGATHER):
        drain(r)

    lanes = lax.iota(jnp.int32, 16)
    m9 = jnp.full((16,), 511, jnp.int32)

    def body(g, _):
        accs = [jnp.zeros((16,), jnp.float32) for _ in range(3)]
        for j in range(NSLOT):
            jb = j * B_PER_W + g * 16
            v0 = cid_v[pl.ds(jb, 16)]
            v1 = cid_v[pl.ds(PAIRS_PER_W + jb, 16)]
            cids = [
                v0 & m9,
                (v0 >> 9) & m9,
                v0 >> 18,
                v1 & m9,
                v1 >> 9,
            ]
            for cid in cids:
                taddr = cid * NCOL
                for k in range(3):
                    accs[k] = accs[k] + plsc.load_gather(t_v, [taddr + (j * 3 + k)])
        m = jnp.maximum(accs[0], jnp.maximum(accs[1], accs[2]))
        outrow = g * 16 + lanes
        for k in range(3):
            plsc.store_scatter(
                out_v, [outrow, jnp.full((16,), k, jnp.int32)], jnp.exp(accs[k] - m)
            )
        return 0

    lax.fori_loop(0, GROUPS, body, 0)

    pltpu.sync_copy(out_v, out_hbm.at[pl.ds(wid * B_PER_W, B_PER_W)])


@jax.jit
def kernel(char_ids, buffer_idx, stack_idx, char_table, W, b):
    t_tab = _build_fold_table(char_table, W, b)
    ct5 = char_ids.astype(jnp.int32).T
    p0 = ct5[0] | (ct5[1] << 9) | (ct5[2] << 18)
    p1 = ct5[3] | (ct5[4] << 9)
    bidx_r = buffer_idx.astype(jnp.int32).T.reshape(3 * B // CHUNK, CHUNK)
    sidx_r = stack_idx.astype(jnp.int32).T.reshape(3 * B // CHUNK, CHUNK)

    mesh = plsc.VectorSubcoreMesh(core_axis_name="c", subcore_axis_name="s")
    sc_params = pltpu.CompilerParams(
        needs_layout_passes=False, use_tc_tiling_on_sc=False
    )
    run = functools.partial(
        pl.kernel,
        mesh=mesh,
        out_type=jax.ShapeDtypeStruct((B, 3), jnp.float32),
        scratch_types=[
            pltpu.VMEM((NCHUNK, CHUNK), jnp.int32),
            pltpu.VMEM((NPLANE * PAIRS_PER_W,), jnp.int32),
            pltpu.VMEM((500 * NCOL,), jnp.float32),
            pltpu.VMEM((B_PER_W, 3), jnp.float32),
            pltpu.SemaphoreType.DMA,
        ],
        compiler_params=sc_params,
    )(_sc_kernel)
    return run(bidx_r, sidx_r, p0, p1, t_tab)


# speedup vs baseline: 11.8178x; 1.1204x over previous
import functools

import jax
import jax.numpy as jnp
from jax import lax
from jax.experimental import pallas as pl
from jax.experimental.pallas import tpu as pltpu
from jax.experimental.pallas import tpu_sc as plsc

B = 16384
N_WORDS = 327680
H = 50
NSLOT = 6
NCHAR = 5
NCOL = NSLOT * 3

NW = 32
B_PER_W = B // NW
PAIRS_PER_W = B_PER_W * NSLOT
CHUNK = 128
NCHUNK = PAIRS_PER_W // CHUNK
ROWS_PER_SLOT = B_PER_W // CHUNK
NPLANE = 2
NGATHER = NPLANE * NCHUNK
DEPTH = 16
GROUPS = B_PER_W // 16


def _fold_table_kernel(ctt_ref, wt_ref, b_ref, out_ref):
    ctt = ctt_ref[...]
    for j in range(NSLOT):
        wj = wt_ref[:, pl.ds(j * H, H)]
        blk = jax.lax.dot_general(
            ctt, wj, (((0,), (1,)), ((), ())),
            preferred_element_type=jnp.float32,
        )
        out_ref[:, pl.ds(j * 3, 3)] = blk + b_ref[...] * (1.0 / (NSLOT * NCHAR))


def _build_fold_table(char_table, W, b):
    t_tab = pl.pallas_call(
        _fold_table_kernel,
        out_shape=jax.ShapeDtypeStruct((500, NCOL), jnp.float32),
    )(char_table.T, W.T, b)
    return t_tab.reshape(-1)


def _sc_kernel(bidx_hbm, sidx_hbm, p0_hbm, p1_hbm, t_hbm, out_hbm,
               idx_v, cid_v, t_v, out_v, sem):
    nc = 2
    wid = lax.axis_index("s") * nc + lax.axis_index("c")

    for j in range(3):
        pltpu.sync_copy(
            bidx_hbm.at[pl.ds(j * CHUNK + wid * ROWS_PER_SLOT, ROWS_PER_SLOT)],
            idx_v.at[pl.ds(j * ROWS_PER_SLOT, ROWS_PER_SLOT)],
        )
        pltpu.sync_copy(
            sidx_hbm.at[pl.ds(j * CHUNK + wid * ROWS_PER_SLOT, ROWS_PER_SLOT)],
            idx_v.at[pl.ds((j + 3) * ROWS_PER_SLOT, ROWS_PER_SLOT)],
        )

    pltpu.sync_copy(t_hbm, t_v)

    def start(r):
        plane, q = divmod(r, NCHUNK)
        src = p0_hbm if plane == 0 else p1_hbm
        return pltpu.async_copy(
            src.at[idx_v.at[q]],
            cid_v.at[pl.ds(r * CHUNK, CHUNK)],
            sem,
        )

    def drain(r):
        plane, q = divmod(r, NCHUNK)
        src = p0_hbm if plane == 0 else p1_hbm
        pltpu.make_async_copy(
            src.at[idx_v.at[q]], cid_v.at[pl.ds(r * CHUNK, CHUNK)], sem
        ).wait()

    for r in range(DEPTH):
        start(r)
    for r in range(DEPTH, NGATHER):
        start(r)
        drain(r - DEPTH)
    for r in range(NGATHER - DEPTH, NGATHER):
        drain(r)

    lanes = lax.iota(jnp.int32, 16)
    m9 = jnp.full((16,), 511, jnp.int32)

    def body(g, _):
        accs = [jnp.zeros((16,), jnp.float32) for _ in range(3)]
        for j in range(NSLOT):
            jb = j * B_PER_W + g * 16
            v0 = cid_v[pl.ds(jb, 16)]
            v1 = cid_v[pl.ds(PAIRS_PER_W + jb, 16)]
            cids = [
                v0 & m9,
                (v0 >> 9) & m9,
                v0 >> 18,
                v1 & m9,
                v1 >> 9,
            ]
            for cid in cids:
                taddr = cid * NCOL
                for k in range(3):
                    accs[k] = accs[k] + plsc.load_gather(t_v, [taddr + (j * 3 + k)])
        m = jnp.maximum(accs[0], jnp.maximum(accs[1], accs[2]))
        outrow = g * 16 + lanes
        for k in range(3):
            plsc.store_scatter(
                out_v, [outrow, jnp.full((16,), k, jnp.int32)], jnp.exp(accs[k] - m)
            )
        return 0

    lax.fori_loop(0, GROUPS, body, 0)

    pltpu.sync_copy(out_v, out_hbm.at[pl.ds(wid * B_PER_W, B_PER_W)])


@jax.jit
def kernel(char_ids, buffer_idx, stack_idx, char_table, W, b):
    t_tab = _build_fold_table(char_table, W, b)
    cid_lin = char_ids.astype(jnp.int32).T.reshape(-1)
    c0, c1, c2, c3, c4 = (
        lax.dynamic_slice_in_dim(cid_lin, c * N_WORDS, N_WORDS) for c in range(5)
    )
    p0 = c0 | (c1 << 9) | (c2 << 18)
    p1 = c3 | (c4 << 9)
    bidx_r = buffer_idx.astype(jnp.int32).T.reshape(3 * B // CHUNK, CHUNK)
    sidx_r = stack_idx.astype(jnp.int32).T.reshape(3 * B // CHUNK, CHUNK)

    mesh = plsc.VectorSubcoreMesh(core_axis_name="c", subcore_axis_name="s")
    sc_params = pltpu.CompilerParams(
        needs_layout_passes=False, use_tc_tiling_on_sc=False
    )
    run = functools.partial(
        pl.kernel,
        mesh=mesh,
        out_type=jax.ShapeDtypeStruct((B, 3), jnp.float32),
        scratch_types=[
            pltpu.VMEM((NCHUNK, CHUNK), jnp.int32),
            pltpu.VMEM((NPLANE * PAIRS_PER_W,), jnp.int32),
            pltpu.VMEM((500 * NCOL,), jnp.float32),
            pltpu.VMEM((B_PER_W, 3), jnp.float32),
            pltpu.SemaphoreType.DMA,
        ],
        compiler_params=sc_params,
    )(_sc_kernel)
    return run(bidx_r, sidx_r, p0, p1, t_tab)


# speedup vs baseline: 19.0302x; 1.6103x over previous
import functools

import jax
import jax.numpy as jnp
from jax import lax
from jax.experimental import pallas as pl
from jax.experimental.pallas import tpu as pltpu
from jax.experimental.pallas import tpu_sc as plsc

B = 16384
N_WORDS = 327680
H = 50
NSLOT = 6
NCHAR = 5
NCOL = NSLOT * 3

NW = 32
B_PER_W = B // NW
PAIRS_PER_W = B_PER_W * NSLOT
CHUNK = 128
NCHUNK = PAIRS_PER_W // CHUNK
ROWS_PER_SLOT = B_PER_W // CHUNK
NPLANE = 2
GROUPS = B_PER_W // 16


def _fold_table_kernel(ctt_ref, wt_ref, b_ref, out_ref):
    ctt = ctt_ref[...]
    for j in range(NSLOT):
        wj = wt_ref[:, pl.ds(j * H, H)]
        blk = jax.lax.dot_general(
            ctt, wj, (((0,), (1,)), ((), ())),
            preferred_element_type=jnp.float32,
        )
        out_ref[:, pl.ds(j * 3, 3)] = blk + b_ref[...] * (1.0 / (NSLOT * NCHAR))


def _build_fold_table(char_table, W, b):
    t_tab = pl.pallas_call(
        _fold_table_kernel,
        out_shape=jax.ShapeDtypeStruct((500, NCOL), jnp.float32),
    )(char_table.T, W.T, b)
    return t_tab.reshape(-1)


PACK_BLK = 65536


def _pack_kernel(ct5_ref, p0_ref, p1_ref):
    rows = [ct5_ref[c] for c in range(NCHAR)]
    p0_ref[...] = rows[0] | (rows[1] << 9) | (rows[2] << 18)
    p1_ref[...] = rows[3] | (rows[4] << 9)


def _pack_planes(char_ids):
    ct5 = char_ids.astype(jnp.int32).T
    return pl.pallas_call(
        _pack_kernel,
        grid=(N_WORDS // PACK_BLK,),
        in_specs=[pl.BlockSpec((NCHAR, PACK_BLK), lambda i: (0, i))],
        out_specs=[
            pl.BlockSpec((PACK_BLK,), lambda i: (i,)),
            pl.BlockSpec((PACK_BLK,), lambda i: (i,)),
        ],
        out_shape=[
            jax.ShapeDtypeStruct((N_WORDS,), jnp.int32),
            jax.ShapeDtypeStruct((N_WORDS,), jnp.int32),
        ],
    )(ct5)


def _sc_kernel(bidx_hbm, sidx_hbm, p0_hbm, p1_hbm, t_hbm, out_hbm,
               idx_v, cid_v, t_v, out_v, sem_stage, s0, s1, s2, s3):
    nc = 2
    wid = lax.axis_index("s") * nc + lax.axis_index("c")
    stripe_sems = [s0, s1, s2, s3]

    stage = []
    for j in range(3):
        stage.append(pltpu.async_copy(
            bidx_hbm.at[pl.ds(j * CHUNK + wid * ROWS_PER_SLOT, ROWS_PER_SLOT)],
            idx_v.at[pl.ds(j * ROWS_PER_SLOT, ROWS_PER_SLOT)],
            sem_stage,
        ))
        stage.append(pltpu.async_copy(
            sidx_hbm.at[pl.ds(j * CHUNK + wid * ROWS_PER_SLOT, ROWS_PER_SLOT)],
            idx_v.at[pl.ds((j + 3) * ROWS_PER_SLOT, ROWS_PER_SLOT)],
            sem_stage,
        ))
    t_copy = pltpu.async_copy(t_hbm, t_v, sem_stage)
    for c in stage:
        c.wait()

    def stripe_chunks(s):
        out = []
        for plane in range(NPLANE):
            src = p0_hbm if plane == 0 else p1_hbm
            for j in range(NSLOT):
                q = 4 * j + s
                r = plane * NCHUNK + q
                out.append((src, q, r))
        return out

    def fire(s):
        for src, q, r in stripe_chunks(s):
            pltpu.async_copy(
                src.at[idx_v.at[q]], cid_v.at[pl.ds(r * CHUNK, CHUNK)],
                stripe_sems[s],
            )

    def drain(s):
        for src, q, r in stripe_chunks(s):
            pltpu.make_async_copy(
                src.at[idx_v.at[q]], cid_v.at[pl.ds(r * CHUNK, CHUNK)],
                stripe_sems[s],
            ).wait()

    m9 = jnp.full((16,), 511, jnp.int32)

    def body(g, _):
        accs = [jnp.zeros((16,), jnp.float32) for _ in range(3)]
        for j in range(NSLOT):
            jb = j * B_PER_W + g * 16
            v0 = cid_v[pl.ds(jb, 16)]
            v1 = cid_v[pl.ds(PAIRS_PER_W + jb, 16)]
            cids = [
                v0 & m9,
                (v0 >> 9) & m9,
                v0 >> 18,
                v1 & m9,
                v1 >> 9,
            ]
            for cid in cids:
                taddr = cid * NCOL
                for k in range(3):
                    accs[k] = accs[k] + plsc.load_gather(t_v, [taddr + (j * 3 + k)])
        m = jnp.maximum(accs[0], jnp.maximum(accs[1], accs[2]))
        for k in range(3):
            out_v[k, pl.ds(g * 16, 16)] = jnp.exp(accs[k] - m)
        return 0

    fire(0)
    fire(1)
    t_copy.wait()
    for s in range(4):
        if s + 2 < 4:
            fire(s + 2)
        drain(s)
        lax.fori_loop(8 * s, 8 * s + 8, body, 0)

    for k in range(3):
        pltpu.sync_copy(
            out_v.at[k], out_hbm.at[k, pl.ds(wid * B_PER_W, B_PER_W)]
        )


@jax.jit
def kernel(char_ids, buffer_idx, stack_idx, char_table, W, b):
    t_tab = _build_fold_table(char_table, W, b)
    p0, p1 = _pack_planes(char_ids)
    bidx_r = buffer_idx.astype(jnp.int32).T.reshape(3 * B // CHUNK, CHUNK)
    sidx_r = stack_idx.astype(jnp.int32).T.reshape(3 * B // CHUNK, CHUNK)

    mesh = plsc.VectorSubcoreMesh(core_axis_name="c", subcore_axis_name="s")
    sc_params = pltpu.CompilerParams(
        needs_layout_passes=False, use_tc_tiling_on_sc=False
    )
    run = functools.partial(
        pl.kernel,
        mesh=mesh,
        out_type=jax.ShapeDtypeStruct((3, B), jnp.float32),
        scratch_types=[
            pltpu.VMEM((NCHUNK, CHUNK), jnp.int32),
            pltpu.VMEM((NPLANE * PAIRS_PER_W,), jnp.int32),
            pltpu.VMEM((500 * NCOL,), jnp.float32),
            pltpu.VMEM((3, B_PER_W), jnp.float32),
            pltpu.SemaphoreType.DMA,
            pltpu.SemaphoreType.DMA,
            pltpu.SemaphoreType.DMA,
            pltpu.SemaphoreType.DMA,
            pltpu.SemaphoreType.DMA,
        ],
        compiler_params=sc_params,
    )(_sc_kernel)
    return run(bidx_r, sidx_r, p0, p1, t_tab).T
